# Initial kernel scaffold; baseline (speedup 1.0000x reference)
import jax
import jax.numpy as jnp
from jax.experimental import pallas as pl


def kernel(x, indices):
    raise NotImplementedError("write your pallas kernel here")


# speedup vs baseline: 2.6410x; 2.6410x over previous
import functools

import jax
import jax.numpy as jnp
from jax import lax
from jax.experimental import pallas as pl
from jax.experimental.pallas import tpu as pltpu
from jax.experimental.pallas import tpu_sc as plsc

B, H, W, C = 4, 256, 256, 96
OUT_SZ = (2 * H) * (2 * W) * C
N = B * H * W * C
TOT_OUT = B * OUT_SZ

R = 3 * 2**19
NREG = 64
REG_PER_SC = NREG // 2
TOT_PAD = NREG * R

NTILE = 16
CH = N // NTILE
WS = 8192
NSUB = CH // WS
SLICE = R // NTILE
ZCH = 4096

_SH_PAD = 256


def _decode_body(idx_ref, g_ref):
    b = pl.program_id(0)
    idx = idx_ref[...]
    q = idx >> 23
    k = (q * 11) >> 5
    off = idx - k * OUT_SZ
    g_ref[...] = off + b * OUT_SZ


def _decode(indices):
    return pl.pallas_call(
        _decode_body,
        out_shape=jax.ShapeDtypeStruct((B, H, W, C), jnp.int32),
        grid=(B, H),
        in_specs=[pl.BlockSpec((1, 1, W, C), lambda b, h: (b, h, 0, 0))],
        out_specs=pl.BlockSpec((1, 1, W, C), lambda b, h: (b, h, 0, 0)),
    )(indices)


def _sc_body(g_hbm, x_hbm, out_hbm, gbuf, xbuf, ibuf, zbuf, sh):
    sc = lax.axis_index("c")
    tile = lax.axis_index("s")
    lane = lax.iota(jnp.int32, 16)
    dump = R + tile * 16 + lane
    zero16 = jnp.zeros((16,), jnp.float32)

    def zfill(i, _):
        zbuf[pl.ds(i * 16, 16)] = zero16
        return 0

    lax.fori_loop(0, ZCH // 16, zfill, 0)

    def region(r, _):
        lo = (sc * REG_PER_SC + r) * R

        def zslice(i, _):
            pltpu.sync_copy(zbuf, sh.at[pl.ds(tile * SLICE + i * ZCH, ZCH)])
            return 0

        lax.fori_loop(0, SLICE // ZCH, zslice, 0)
        plsc.subcore_barrier()

        def subw(w, _):
            base = tile * CH + w * WS
            pltpu.sync_copy(g_hbm.at[pl.ds(base, WS)], gbuf)
            pltpu.sync_copy(x_hbm.at[pl.ds(base, WS)], xbuf)

            def octet(i, _):
                for u in range(8):
                    sl = pl.ds((i * 8 + u) * 16, 16)
                    t = gbuf[sl] - lo
                    m = plsc.bitcast(t, jnp.uint32) < jnp.uint32(R)
                    ibuf[sl] = jnp.where(m, t, dump)
                return 0

            lax.fori_loop(0, WS // 128, octet, 0)
            pltpu.sync_copy(xbuf, sh.at[ibuf], add=True)
            return 0

        lax.fori_loop(0, NSUB, subw, 0)
        plsc.subcore_barrier()
        pltpu.sync_copy(
            sh.at[pl.ds(tile * SLICE, SLICE)],
            out_hbm.at[pl.ds(lo + tile * SLICE, SLICE)],
        )
        return 0

    lax.fori_loop(0, REG_PER_SC, region, 0)


@jax.jit
def kernel(x, indices):
    g = _decode(indices).reshape(N)
    xf = x.reshape(N)
    mesh = plsc.VectorSubcoreMesh(core_axis_name="c", subcore_axis_name="s")
    scatter = pl.kernel(
        _sc_body,
        out_type=jax.ShapeDtypeStruct((TOT_PAD,), jnp.float32),
        mesh=mesh,
        scratch_types=[
            pltpu.VMEM((WS,), jnp.int32),
            pltpu.VMEM((WS,), jnp.float32),
            pltpu.VMEM((WS,), jnp.int32),
            pltpu.VMEM((ZCH,), jnp.float32),
            pltpu.VMEM_SHARED((R + _SH_PAD,), jnp.float32),
        ],
    )
    out_pad = scatter(g, xf)
    return out_pad[:TOT_OUT].reshape(B, 2 * H, 2 * W, C)


# speedup vs baseline: 4.6829x; 1.7731x over previous
import jax
import jax.numpy as jnp
from jax import lax
from jax.experimental import pallas as pl
from jax.experimental.pallas import tpu as pltpu
from jax.experimental.pallas import tpu_sc as plsc

B, H, W, C = 4, 256, 256, 96
OUT_SZ = (2 * H) * (2 * W) * C
N = B * H * W * C
TOT_OUT = B * OUT_SZ

R = 3 * 2**19
NREG = 64
REG_PER_SC = NREG // 2

NTILE = 16
CH = N // NTILE
WS = 2048
NSUB = CH // WS
NBUF = 4
SLICE = R // NTILE
ZCH = 4096

_SH_PAD = 256


def _decode_body(idx_ref, g_ref):
    b = pl.program_id(0)
    idx = idx_ref[...]
    q = idx >> 23
    k = (q * 11) >> 5
    off = idx - k * OUT_SZ
    g_ref[...] = off + b * OUT_SZ


def _decode(indices):
    return pl.pallas_call(
        _decode_body,
        out_shape=jax.ShapeDtypeStruct((B, H, W, C), jnp.int32),
        grid=(B, H),
        in_specs=[pl.BlockSpec((1, 1, W, C), lambda b, h: (b, h, 0, 0))],
        out_specs=pl.BlockSpec((1, 1, W, C), lambda b, h: (b, h, 0, 0)),
    )(indices)


def _sc_body(g_hbm, x_hbm, out_hbm, *refs):
    gb = refs[0:NBUF]
    xb = refs[NBUF:2 * NBUF]
    ib = refs[2 * NBUF:3 * NBUF]
    zbuf = refs[3 * NBUF]
    sh = refs[3 * NBUF + 1]
    sg = refs[3 * NBUF + 2:3 * NBUF + 2 + NBUF]
    sx = refs[3 * NBUF + 2 + NBUF:3 * NBUF + 2 + 2 * NBUF]
    sa = refs[3 * NBUF + 2 + 2 * NBUF:3 * NBUF + 2 + 3 * NBUF]

    sc = lax.axis_index("c")
    tile = lax.axis_index("s")
    lane = lax.iota(jnp.int32, 16)
    dump = R + tile * 16 + lane
    zero16 = jnp.zeros((16,), jnp.float32)
    cbase = tile * CH

    def zfill(i, _):
        zbuf[pl.ds(i * 16, 16)] = zero16
        return 0

    lax.fori_loop(0, ZCH // 16, zfill, 0)

    def pf_issue(w, p):
        base = cbase + w * WS
        pltpu.async_copy(g_hbm.at[pl.ds(base, WS)], gb[p], sg[p])
        pltpu.async_copy(x_hbm.at[pl.ds(base, WS)], xb[p], sx[p])

    def pf_wait(p):
        pltpu.make_async_copy(g_hbm.at[pl.ds(0, WS)], gb[p], sg[p]).wait()
        pltpu.make_async_copy(x_hbm.at[pl.ds(0, WS)], xb[p], sx[p]).wait()

    def add_wait(p):
        pltpu.make_async_copy(xb[p], sh.at[ib[p]], sa[p]).wait()

    def region(r, _):
        lo = (sc * REG_PER_SC + r) * R

        def zslice(i, _):
            pltpu.sync_copy(zbuf, sh.at[pl.ds(tile * SLICE + i * ZCH, ZCH)])
            return 0

        lax.fori_loop(0, SLICE // ZCH, zslice, 0)
        plsc.subcore_barrier()

        pf_issue(0, 0)
        pf_issue(1, 1)

        def quad(i, _):
            for u in range(NBUF):
                p = u
                w = i * NBUF + u
                pf_wait(p)

                def octet(j, _):
                    for v in range(8):
                        sl = pl.ds((j * 8 + v) * 16, 16)
                        t = gb[p][sl] - lo
                        m = plsc.bitcast(t, jnp.uint32) < jnp.uint32(R)
                        ib[p][sl] = jnp.where(m, t, dump)
                    return 0

                lax.fori_loop(0, WS // 128, octet, 0)
                pltpu.async_copy(xb[p], sh.at[ib[p]], sa[p], add=True)

                p2 = (u + 2) % NBUF
                if u < 2:
                    @pl.when(i >= 1)
                    def _():
                        add_wait(p2)

                    pf_issue(w + 2, p2)
                else:
                    add_wait(p2)

                    @pl.when(i < NSUB // NBUF - 1)
                    def _():
                        pf_issue(w + 2, p2)
            return 0

        lax.fori_loop(0, NSUB // NBUF, quad, 0)
        add_wait((NSUB - 2) % NBUF)
        add_wait((NSUB - 1) % NBUF)
        plsc.subcore_barrier()
        pltpu.sync_copy(
            sh.at[pl.ds(tile * SLICE, SLICE)],
            out_hbm.at[pl.ds(lo + tile * SLICE, SLICE)],
        )
        return 0

    lax.fori_loop(0, REG_PER_SC, region, 0)


@jax.jit
def kernel(x, indices):
    g = _decode(indices).reshape(N)
    xf = x.reshape(N)
    mesh = plsc.VectorSubcoreMesh(core_axis_name="c", subcore_axis_name="s")
    scratch = (
        [pltpu.VMEM((WS,), jnp.int32) for _ in range(NBUF)]
        + [pltpu.VMEM((WS,), jnp.float32) for _ in range(NBUF)]
        + [pltpu.VMEM((WS,), jnp.int32) for _ in range(NBUF)]
        + [pltpu.VMEM((ZCH,), jnp.float32)]
        + [pltpu.VMEM_SHARED((R + _SH_PAD,), jnp.float32)]
        + [pltpu.SemaphoreType.DMA for _ in range(3 * NBUF)]
    )
    scatter = pl.kernel(
        _sc_body,
        out_type=jax.ShapeDtypeStruct((TOT_OUT,), jnp.float32),
        mesh=mesh,
        scratch_types=scratch,
    )
    out = scatter(g, xf)
    return out.reshape(B, 2 * H, 2 * W, C)
